# Initial kernel scaffold; baseline (speedup 1.0000x reference)
import jax
import jax.numpy as jnp
from jax.experimental import pallas as pl


def kernel(inputs, adj, mask_item, item, adj_all, num, embedding, a0, a1, a2, a3, g_w1, g_w2, g_w3, g_bias):
    raise NotImplementedError("write your pallas kernel here")


# speedup vs baseline: 1.7929x; 1.7929x over previous
import functools

import jax
import jax.numpy as jnp
import numpy as np
from jax import lax
from jax.experimental import pallas as pl
from jax.experimental.pallas import tpu as pltpu
from jax.experimental.pallas import tpu_sc as plsc

B, L, D = 256, 50, 64
S = 12
BL = B * L
LS = L * S
ALPHA = 0.2


def _sc_gather(inp_flat, item_flat, pos_flat, adj_flat, num, embedding):
    info = plsc.get_sparse_core_info()
    nc, ns = info.num_cores, info.num_subcores
    nw = nc * ns
    per_w = BL // nw
    ch = 80
    n_chunks = per_w // ch
    mesh = plsc.VectorSubcoreMesh(core_axis_name="c", subcore_axis_name="s")

    @functools.partial(
        pl.kernel,
        out_type=(
            jax.ShapeDtypeStruct((BL, D), jnp.float32),
            jax.ShapeDtypeStruct((BL, S), jnp.float32),
            jax.ShapeDtypeStruct((BL, D), jnp.float32),
            jax.ShapeDtypeStruct((BL * S, D), jnp.float32),
        ),
        mesh=mesh,
        compiler_params=pltpu.CompilerParams(use_tc_tiling_on_sc=False),
        scratch_types=[
            pltpu.VMEM((ch,), jnp.int32),
            pltpu.VMEM((ch, D), jnp.float32),
            pltpu.VMEM((ch * S,), jnp.int32),
            pltpu.VMEM((ch, S), jnp.float32),
            pltpu.VMEM((ch * S,), jnp.int32),
            pltpu.VMEM((ch * S, D), jnp.float32),
            pltpu.VMEM((ch,), jnp.int32),
            pltpu.VMEM((ch, D), jnp.float32),
            pltpu.SemaphoreType.DMA,
            pltpu.SemaphoreType.DMA,
            pltpu.SemaphoreType.DMA,
            pltpu.SemaphoreType.DMA,
        ],
    )
    def gather_kernel(inp_hbm, item_hbm, pos_hbm, adjflat_hbm, num_hbm,
                      emb_hbm,
                      ent0_hbm, nw_hbm, item_emb_hbm, ent1_hbm,
                      idx_v, rows_v, pos_v, num_v, nidx_v, nrows_v,
                      iidx_v, irows_v, sem0, sem1, sem2, sem3):
        wid = lax.axis_index("s") * nc + lax.axis_index("c")
        base = wid * per_w
        for c in range(n_chunks):
            off = base + c * ch
            pltpu.sync_copy(inp_hbm.at[pl.ds(off, ch)], idx_v)
            pltpu.sync_copy(item_hbm.at[pl.ds(off, ch)], iidx_v)
            pltpu.sync_copy(pos_hbm.at[pl.ds(off * S, ch * S)], pos_v)
            cp_nid = pltpu.async_copy(adjflat_hbm.at[pos_v], nidx_v, sem1)
            cp_e0 = pltpu.async_copy(emb_hbm.at[idx_v], rows_v, sem0)
            cp_nw = pltpu.async_copy(num_hbm.at[idx_v], num_v, sem2)
            cp_it = pltpu.async_copy(emb_hbm.at[iidx_v], irows_v, sem3)
            cp_nid.wait()
            cp_e1 = pltpu.async_copy(emb_hbm.at[nidx_v], nrows_v, sem1)
            cp_e0.wait()
            pltpu.sync_copy(rows_v, ent0_hbm.at[pl.ds(off, ch)])
            cp_nw.wait()
            pltpu.sync_copy(num_v, nw_hbm.at[pl.ds(off, ch)])
            cp_it.wait()
            pltpu.sync_copy(irows_v, item_emb_hbm.at[pl.ds(off, ch)])
            cp_e1.wait()
            pltpu.sync_copy(nrows_v, ent1_hbm.at[pl.ds(off * S, ch * S)])

    return gather_kernel(inp_flat, item_flat, pos_flat, adj_flat, num,
                         embedding)


def _dot(x, y):
    return lax.dot_general(x, y, (((1,), (0,)), ((), ())),
                           preferred_element_type=jnp.float32)


def _dot_nt(x, y):
    return lax.dot_general(x, y, (((1,), (1,)), ((), ())),
                           preferred_element_type=jnp.float32)


def _leaky(x, slope):
    return jnp.where(x >= 0, x, slope * x)


def _tc_body(e0_ref, e1_ref, nw_ref, it_ref, adj_ref, mf_ref,
             a4_ref, w1a_ref, w1b_ref, w2_ref, w3a_ref, w3b_ref, bias_ref,
             out_ref, *, bb):
    a4 = a4_ref[...]
    w1a = w1a_ref[...]
    w1b = w1b_ref[...]
    w2 = w2_ref[...]
    w3a = w3a_ref[...]
    w3b = w3b_ref[...]
    bias = bias_ref[...]
    rows = lax.broadcasted_iota(jnp.int32, (L, LS), 0)
    cols = lax.broadcasted_iota(jnp.int32, (L, LS), 1) // S
    msel = (rows == cols).astype(jnp.float32)
    for i in range(bb):
        e0 = e0_ref[i]
        n2 = jnp.sum(e0 * e0, axis=1, keepdims=True)
        h = e0 / jnp.maximum(jnp.sqrt(n2), 1e-12)
        adji = adj_ref[i]
        logits = jnp.full((L, L), -9e15, dtype=jnp.float32)
        for k in range(4):
            ek = _leaky(_dot_nt(h * a4[k:k + 1, :], h), ALPHA)
            logits = jnp.where(adji == k + 1, ek, logits)
        m = jnp.max(logits, axis=1, keepdims=True)
        p = jnp.exp(logits - m)
        aw = p / jnp.sum(p, axis=1, keepdims=True)
        h_local = _dot(aw, h)
        mfi = mf_ref[i]
        sess = (jnp.sum(it_ref[i] * mfi, axis=0, keepdims=True)
                / jnp.sum(mfi))
        e1 = e1_ref[i]
        nwi = nw_ref[i]
        z = _leaky(_dot(e1 * sess, w1a) + nwi * w1b, 0.2)
        ex = jnp.exp(_dot(z, w2))
        acc = _dot(msel, e1 * ex)
        den = _dot(msel, ex)
        neigh = acc / den
        hg = jnp.maximum(_dot(e0, w3a) + _dot(neigh, w3b) + bias, 0.0)
        out_ref[i] = h_local + hg


def _tc_compute(ent0, ent1, nw, item_rows, adj, maskf,
                a4, w1a, w1b, w2, w3a, w3b, bias, bb=4):
    grid = (B // bb,)
    bspec = lambda shape: pl.BlockSpec((bb,) + shape, lambda g: (g,) + (0,) * len(shape))
    wspec = lambda shape: pl.BlockSpec(shape, lambda g: (0,) * len(shape))
    return pl.pallas_call(
        functools.partial(_tc_body, bb=bb),
        grid=grid,
        in_specs=[
            bspec((L, D)),
            bspec((LS, D)),
            bspec((LS, 1)),
            bspec((L, D)),
            bspec((L, L)),
            bspec((L, 1)),
            wspec((4, D)),
            wspec((D, D)),
            wspec((1, D)),
            wspec((D, 1)),
            wspec((D, D)),
            wspec((D, D)),
            wspec((1, D)),
        ],
        out_specs=bspec((L, D)),
        out_shape=jax.ShapeDtypeStruct((B, L, D), jnp.float32),
    )(ent0, ent1, nw, item_rows, adj, maskf, a4, w1a, w1b, w2, w3a, w3b, bias)


def kernel(inputs, adj, mask_item, item, adj_all, num, embedding,
           a0, a1, a2, a3, g_w1, g_w2, g_w3, g_bias):
    inp_flat = inputs.reshape(-1).astype(jnp.int32)
    item_flat = item.reshape(-1).astype(jnp.int32)
    pos_flat = (inp_flat[:, None] * S
                + jnp.arange(S, dtype=jnp.int32)[None, :]).reshape(-1)
    adj_flat = adj_all.astype(jnp.int32).reshape(-1)
    ent0, nwg, item_rows, ent1 = _sc_gather(
        inp_flat, item_flat, pos_flat, adj_flat, num, embedding)
    ent0 = ent0.reshape(B, L, D)
    item_rows = item_rows.reshape(B, L, D)
    ent1 = ent1.reshape(B, LS, D)
    nwg = nwg.reshape(B, LS, 1)
    a4 = jnp.concatenate([a0, a1, a2, a3], axis=1).T
    w1a = g_w1[:D]
    w1b = g_w1[D:]
    w3a = g_w3[:D]
    w3b = g_w3[D:]
    bias = g_bias.reshape(1, D)
    maskf = mask_item.astype(jnp.float32).reshape(B, L, 1)
    return _tc_compute(ent0, ent1, nwg, item_rows, adj, maskf,
                       a4, w1a, w1b, g_w2, w3a, w3b, bias)


# speedup vs baseline: 1.8778x; 1.0474x over previous
import functools

import jax
import jax.numpy as jnp
import numpy as np
from jax import lax
from jax.experimental import pallas as pl
from jax.experimental.pallas import tpu as pltpu
from jax.experimental.pallas import tpu_sc as plsc

B, L, D = 256, 50, 64
S = 12
BL = B * L
LS = L * S
ALPHA = 0.2


def _sc_gather(inp_flat, item_flat, pos_flat, adj_flat, num, embedding):
    info = plsc.get_sparse_core_info()
    nc, ns = info.num_cores, info.num_subcores
    nw = nc * ns
    per_w = BL // nw
    ch = 80
    n_chunks = per_w // ch
    mesh = plsc.VectorSubcoreMesh(core_axis_name="c", subcore_axis_name="s")

    @functools.partial(
        pl.kernel,
        out_type=(
            jax.ShapeDtypeStruct((BL, D), jnp.float32),
            jax.ShapeDtypeStruct((BL, S), jnp.float32),
            jax.ShapeDtypeStruct((BL, D), jnp.float32),
            jax.ShapeDtypeStruct((BL * S, D), jnp.float32),
        ),
        mesh=mesh,
        compiler_params=pltpu.CompilerParams(use_tc_tiling_on_sc=False),
        scratch_types=[
            pltpu.VMEM((ch,), jnp.int32),
            pltpu.VMEM((ch, D), jnp.float32),
            pltpu.VMEM((ch * S,), jnp.int32),
            pltpu.VMEM((ch, S), jnp.float32),
            pltpu.VMEM((ch * S,), jnp.int32),
            pltpu.VMEM((ch * S, D), jnp.float32),
            pltpu.VMEM((ch,), jnp.int32),
            pltpu.VMEM((ch, D), jnp.float32),
            pltpu.SemaphoreType.DMA,
            pltpu.SemaphoreType.DMA,
            pltpu.SemaphoreType.DMA,
            pltpu.SemaphoreType.DMA,
        ],
    )
    def gather_kernel(inp_hbm, item_hbm, pos_hbm, adjflat_hbm, num_hbm,
                      emb_hbm,
                      ent0_hbm, nw_hbm, item_emb_hbm, ent1_hbm,
                      idx_v, rows_v, pos_v, num_v, nidx_v, nrows_v,
                      iidx_v, irows_v, sem0, sem1, sem2, sem3):
        wid = lax.axis_index("s") * nc + lax.axis_index("c")
        base = wid * per_w
        for c in range(n_chunks):
            off = base + c * ch
            pltpu.sync_copy(inp_hbm.at[pl.ds(off, ch)], idx_v)
            pltpu.sync_copy(item_hbm.at[pl.ds(off, ch)], iidx_v)
            pltpu.sync_copy(pos_hbm.at[pl.ds(off * S, ch * S)], pos_v)
            cp_nid = pltpu.async_copy(adjflat_hbm.at[pos_v], nidx_v, sem1)
            cp_e0 = pltpu.async_copy(emb_hbm.at[idx_v], rows_v, sem0)
            cp_nw = pltpu.async_copy(num_hbm.at[idx_v], num_v, sem2)
            cp_it = pltpu.async_copy(emb_hbm.at[iidx_v], irows_v, sem3)
            cp_nid.wait()
            cp_e1 = pltpu.async_copy(emb_hbm.at[nidx_v], nrows_v, sem1)
            cp_e0.wait()
            pltpu.sync_copy(rows_v, ent0_hbm.at[pl.ds(off, ch)])
            cp_nw.wait()
            pltpu.sync_copy(num_v, nw_hbm.at[pl.ds(off, ch)])
            cp_it.wait()
            pltpu.sync_copy(irows_v, item_emb_hbm.at[pl.ds(off, ch)])
            cp_e1.wait()
            pltpu.sync_copy(nrows_v, ent1_hbm.at[pl.ds(off * S, ch * S)])

    return gather_kernel(inp_flat, item_flat, pos_flat, adj_flat, num,
                         embedding)


def _dot(x, y):
    return lax.dot_general(x, y, (((1,), (0,)), ((), ())),
                           preferred_element_type=jnp.float32)


def _dot_nt(x, y):
    return lax.dot_general(x, y, (((1,), (1,)), ((), ())),
                           preferred_element_type=jnp.float32)


def _leaky(x, slope):
    return jnp.where(x >= 0, x, slope * x)


def _tc_body(e0_ref, e1_ref, nw_ref, it_ref, adjb_ref, mf_ref,
             a4_ref, w1a_ref, w1b_ref, w2_ref, w3a_ref, w3b_ref, bias_ref,
             out_ref, *, bb):
    bl = bb * L
    ns = bb * LS
    a4 = a4_ref[...]
    w1a = w1a_ref[...]
    w1b = w1b_ref[...]
    w2 = w2_ref[...]
    w3a = w3a_ref[...]
    w3b = w3b_ref[...]
    bias = bias_ref[...]
    e0 = e0_ref[...].reshape(bl, D)
    n2 = jnp.sum(e0 * e0, axis=1, keepdims=True)
    h = e0 / jnp.maximum(jnp.sqrt(n2), 1e-12)
    adjb = adjb_ref[0]
    rb = lax.broadcasted_iota(jnp.int32, (bl, bl), 0) // L
    cb = lax.broadcasted_iota(jnp.int32, (bl, bl), 1) // L
    logits = jnp.where(rb == cb, -9e15, -1.8e16).astype(jnp.float32)
    for k in range(4):
        ek = _leaky(_dot_nt(h * a4[k:k + 1, :], h), ALPHA)
        logits = jnp.where(adjb == k + 1, ek, logits)
    m = jnp.max(logits, axis=1, keepdims=True)
    p = jnp.exp(logits - m)
    aw = p / jnp.sum(p, axis=1, keepdims=True)
    h_local = _dot(aw, h)
    mf = mf_ref[...]
    sess = jnp.sum(it_ref[...] * mf, axis=1) / jnp.sum(mf, axis=1)
    e1 = e1_ref[...]
    e1s = (e1 * sess[:, None, :]).reshape(ns, D)
    e1f = e1.reshape(ns, D)
    nwf = nw_ref[...].reshape(ns, 1)
    z = _leaky(_dot(e1s, w1a) + nwf * w1b, 0.2)
    ex = jnp.exp(_dot(z, w2))
    acc = jnp.sum((e1f * ex).reshape(bl, S, D), axis=1)
    den = jnp.sum(ex.reshape(bl, S, 1), axis=1)
    neigh = acc / den
    hg = jnp.maximum(_dot(e0, w3a) + _dot(neigh, w3b) + bias, 0.0)
    out_ref[...] = (h_local + hg).reshape(bb, L, D)


def _tc_compute(ent0, ent1, nw, item_rows, adj_big, maskf,
                a4, w1a, w1b, w2, w3a, w3b, bias, bb=4):
    grid = (B // bb,)
    bspec = lambda shape: pl.BlockSpec((bb,) + shape, lambda g: (g,) + (0,) * len(shape))
    wspec = lambda shape: pl.BlockSpec(shape, lambda g: (0,) * len(shape))
    return pl.pallas_call(
        functools.partial(_tc_body, bb=bb),
        grid=grid,
        in_specs=[
            bspec((L, D)),
            bspec((LS, D)),
            bspec((LS, 1)),
            bspec((L, D)),
            pl.BlockSpec((1, bb * L, bb * L), lambda g: (g, 0, 0)),
            bspec((L, 1)),
            wspec((4, D)),
            wspec((D, D)),
            wspec((1, D)),
            wspec((D, 1)),
            wspec((D, D)),
            wspec((D, D)),
            wspec((1, D)),
        ],
        out_specs=bspec((L, D)),
        out_shape=jax.ShapeDtypeStruct((B, L, D), jnp.float32),
    )(ent0, ent1, nw, item_rows, adj_big, maskf,
      a4, w1a, w1b, w2, w3a, w3b, bias)


def _block_diag_adj(adj, bb):
    g = B // bb
    a5 = adj.reshape(g, bb, L, 1, L)
    eye = jnp.eye(bb, dtype=adj.dtype).reshape(1, bb, 1, bb, 1)
    return (a5 * eye).reshape(g, bb * L, bb * L)


def kernel(inputs, adj, mask_item, item, adj_all, num, embedding,
           a0, a1, a2, a3, g_w1, g_w2, g_w3, g_bias):
    inp_flat = inputs.reshape(-1).astype(jnp.int32)
    item_flat = item.reshape(-1).astype(jnp.int32)
    pos_flat = (inp_flat[:, None] * S
                + jnp.arange(S, dtype=jnp.int32)[None, :]).reshape(-1)
    adj_flat = adj_all.astype(jnp.int32).reshape(-1)
    ent0, nwg, item_rows, ent1 = _sc_gather(
        inp_flat, item_flat, pos_flat, adj_flat, num, embedding)
    ent0 = ent0.reshape(B, L, D)
    item_rows = item_rows.reshape(B, L, D)
    ent1 = ent1.reshape(B, LS, D)
    nwg = nwg.reshape(B, LS, 1)
    a4 = jnp.concatenate([a0, a1, a2, a3], axis=1).T
    w1a = g_w1[:D]
    w1b = g_w1[D:]
    w3a = g_w3[:D]
    w3b = g_w3[D:]
    bias = g_bias.reshape(1, D)
    maskf = mask_item.astype(jnp.float32).reshape(B, L, 1)
    return _tc_compute(ent0, ent1, nwg, item_rows, _block_diag_adj(adj, 4),
                       maskf, a4, w1a, w1b, g_w2, w3a, w3b, bias)


# speedup vs baseline: 2.8674x; 1.5270x over previous
import functools

import jax
import jax.numpy as jnp
import numpy as np
from jax import lax
from jax.experimental import pallas as pl
from jax.experimental.pallas import tpu as pltpu
from jax.experimental.pallas import tpu_sc as plsc

B, L, D = 256, 50, 64
S = 12
NUM_NODE = 50000
BL = B * L
LS = L * S
ALPHA = 0.2


def _sc_gather(inp_flat, item_flat, pos_flat, adj_flat, num_flat, embedding):
    info = plsc.get_sparse_core_info()
    nc, ns = info.num_cores, info.num_subcores
    nw = nc * ns
    per_w = BL // nw
    ch = 80
    n_chunks = per_w // ch
    mesh = plsc.VectorSubcoreMesh(core_axis_name="c", subcore_axis_name="s")

    @functools.partial(
        pl.kernel,
        out_type=(
            jax.ShapeDtypeStruct((BL, D), jnp.float32),
            jax.ShapeDtypeStruct((BL * S,), jnp.float32),
            jax.ShapeDtypeStruct((BL, D), jnp.float32),
            jax.ShapeDtypeStruct((BL * S, D), jnp.float32),
        ),
        mesh=mesh,
        compiler_params=pltpu.CompilerParams(use_tc_tiling_on_sc=False),
        scratch_types=[
            pltpu.VMEM((ch,), jnp.int32),
            pltpu.VMEM((ch, D), jnp.float32),
            pltpu.VMEM((ch * S,), jnp.int32),
            pltpu.VMEM((ch * S,), jnp.float32),
            pltpu.VMEM((ch * S,), jnp.int32),
            pltpu.VMEM((ch * S, D), jnp.float32),
            pltpu.VMEM((ch,), jnp.int32),
            pltpu.VMEM((ch, D), jnp.float32),
            pltpu.SemaphoreType.DMA,
            pltpu.SemaphoreType.DMA,
            pltpu.SemaphoreType.DMA,
            pltpu.SemaphoreType.DMA,
        ],
    )
    def gather_kernel(inp_hbm, item_hbm, pos_hbm, adjflat_hbm, numflat_hbm,
                      emb_hbm,
                      ent0_hbm, nw_hbm, item_emb_hbm, ent1_hbm,
                      idx_v, rows_v, pos_v, nw_v, nidx_v, nrows_v,
                      iidx_v, irows_v, sem0, sem1, sem2, sem3):
        wid = lax.axis_index("s") * nc + lax.axis_index("c")
        base = wid * per_w
        for c in range(n_chunks):
            off = base + c * ch
            pltpu.sync_copy(inp_hbm.at[pl.ds(off, ch)], idx_v)
            pltpu.sync_copy(item_hbm.at[pl.ds(off, ch)], iidx_v)
            pltpu.sync_copy(pos_hbm.at[pl.ds(off * S, ch * S)], pos_v)
            cp_nid = pltpu.async_copy(adjflat_hbm.at[pos_v], nidx_v, sem1)
            cp_e0 = pltpu.async_copy(emb_hbm.at[idx_v], rows_v, sem0)
            cp_nw = pltpu.async_copy(numflat_hbm.at[pos_v], nw_v, sem2)
            cp_it = pltpu.async_copy(emb_hbm.at[iidx_v], irows_v, sem3)
            cp_nid.wait()
            cp_e1 = pltpu.async_copy(emb_hbm.at[nidx_v], nrows_v, sem1)
            cp_e0.wait()
            pltpu.sync_copy(rows_v, ent0_hbm.at[pl.ds(off, ch)])
            cp_nw.wait()
            pltpu.sync_copy(nw_v, nw_hbm.at[pl.ds(off * S, ch * S)])
            cp_it.wait()
            pltpu.sync_copy(irows_v, item_emb_hbm.at[pl.ds(off, ch)])
            cp_e1.wait()
            pltpu.sync_copy(nrows_v, ent1_hbm.at[pl.ds(off * S, ch * S)])

    return gather_kernel(inp_flat, item_flat, pos_flat, adj_flat, num_flat,
                         embedding)


def _dot(x, y):
    return lax.dot_general(x, y, (((1,), (0,)), ((), ())),
                           preferred_element_type=jnp.float32)


def _dot_nt(x, y):
    return lax.dot_general(x, y, (((1,), (1,)), ((), ())),
                           preferred_element_type=jnp.float32)


def _leaky(x, slope):
    return jnp.where(x >= 0, x, slope * x)


def _tc_body(e0_ref, e1_ref, nw_ref, it_ref, adj_ref,
             a4_ref, w1big_ref, w2up_ref, sumg_ref, w3a_ref, w3b_ref,
             bias_ref, out_ref, *, bb):
    bl = bb * L
    a4 = a4_ref[...]
    w1big = w1big_ref[...]
    w2up = w2up_ref[...]
    sumg = sumg_ref[...]
    w3a = w3a_ref[...]
    w3b = w3b_ref[...]
    bias = bias_ref[...]
    e0 = e0_ref[...].reshape(bl, D)
    n2 = jnp.sum(e0 * e0, axis=1, keepdims=True)
    h = e0 / jnp.maximum(jnp.sqrt(n2), 1e-12)
    eks = [_leaky(_dot_nt(h * a4[k:k + 1, :], h), ALPHA) for k in range(4)]
    rows = []
    for i in range(bb):
        adji = adj_ref[i]
        li = jnp.full((L, L), -9e15, dtype=jnp.float32)
        for k in range(4):
            sub = eks[k][i * L:(i + 1) * L, i * L:(i + 1) * L]
            li = jnp.where(adji == k + 1, sub, li)
        pieces = []
        if i:
            pieces.append(jnp.full((L, i * L), -1.8e16, dtype=jnp.float32))
        pieces.append(li)
        if i < bb - 1:
            pieces.append(jnp.full((L, (bb - 1 - i) * L), -1.8e16,
                                   dtype=jnp.float32))
        rows.append(jnp.concatenate(pieces, axis=1) if len(pieces) > 1
                    else pieces[0])
    logits = jnp.concatenate(rows, axis=0) if bb > 1 else rows[0]
    m = jnp.max(logits, axis=1, keepdims=True)
    p = jnp.exp(logits - m)
    aw = p / jnp.sum(p, axis=1, keepdims=True)
    h_local = _dot(aw, h)
    sess = jnp.sum(it_ref[...], axis=1)
    sess_bl = jnp.broadcast_to(sess[:, None, :], (bb, L, D)).reshape(bl, D)
    e1w = e1_ref[...].reshape(bl, S * D)
    sessw = jnp.concatenate([sess_bl] * S, axis=1)
    nw12 = nw_ref[...].reshape(bl, S)
    zin = jnp.concatenate([e1w * sessw, nw12], axis=1)
    zl = _leaky(_dot(zin, w1big), 0.2)
    exu = jnp.exp(_dot(zl, w2up))
    accu = _dot(e1w * exu, sumg)
    denu = _dot(exu, sumg)
    neigh = accu / denu
    hg = jnp.maximum(_dot(e0, w3a) + _dot(neigh, w3b) + bias, 0.0)
    out_ref[...] = (h_local + hg).reshape(bb, L, D)


def _tc_compute(ent0, ent1, nw, item_rows, adj,
                a4, w1big, w2up, sumg, w3a, w3b, bias, bb=4):
    grid = (B // bb,)
    bspec = lambda shape: pl.BlockSpec((bb,) + shape, lambda g: (g,) + (0,) * len(shape))
    wspec = lambda shape: pl.BlockSpec(shape, lambda g: (0,) * len(shape))
    return pl.pallas_call(
        functools.partial(_tc_body, bb=bb),
        grid=grid,
        in_specs=[
            bspec((L, D)),
            bspec((L, S * D)),
            bspec((L, S)),
            bspec((L, D)),
            bspec((L, L)),
            wspec((4, D)),
            wspec((S * D + S, S * D)),
            wspec((S * D, S * D)),
            wspec((S * D, D)),
            wspec((D, D)),
            wspec((D, D)),
            wspec((1, D)),
        ],
        out_specs=bspec((L, D)),
        out_shape=jax.ShapeDtypeStruct((B, L, D), jnp.float32),
    )(ent0, ent1, nw, item_rows, adj,
      a4, w1big, w2up, sumg, w3a, w3b, bias)


def kernel(inputs, adj, mask_item, item, adj_all, num, embedding,
           a0, a1, a2, a3, g_w1, g_w2, g_w3, g_bias):
    inp_flat = inputs.reshape(-1).astype(jnp.int32)
    item_flat = item.reshape(-1).astype(jnp.int32)
    pos_flat = (inp_flat[:, None]
                + jnp.arange(S, dtype=jnp.int32)[None, :] * NUM_NODE
                ).reshape(-1)
    adj_t = adj_all.astype(jnp.int32).T.reshape(-1)
    num_t = num.T.reshape(-1)
    ent0, nwg, item_rows, ent1 = _sc_gather(
        inp_flat, item_flat, pos_flat, adj_t, num_t, embedding)
    ent0 = ent0.reshape(B, L, D)
    ent1 = ent1.reshape(B, L, S * D)
    nwg = nwg.reshape(B, L, S)
    maskf = mask_item.astype(jnp.float32)
    scale = (maskf / jnp.sum(maskf, axis=1, keepdims=True)).reshape(BL, 1)
    item_rows = (item_rows * scale).reshape(B, L, D)
    a4 = jnp.concatenate([a0, a1, a2, a3], axis=1).T
    w1a = g_w1[:D]
    w1b = g_w1[D:]
    eye_s = jnp.eye(S, dtype=jnp.float32)
    w1top = (eye_s[:, None, :, None] * w1a[None, :, None, :]
             ).reshape(S * D, S * D)
    w1bot = (eye_s[:, :, None] * w1b[0][None, None, :]).reshape(S, S * D)
    w1big = jnp.concatenate([w1top, w1bot], axis=0)
    w2up = jnp.broadcast_to(
        eye_s[:, None, :, None] * g_w2[:, 0][None, :, None, None],
        (S, D, S, D)).reshape(S * D, S * D)
    sumg = jnp.tile(jnp.eye(D, dtype=jnp.float32), (S, 1))
    w3a = g_w3[:D]
    w3b = g_w3[D:]
    bias = g_bias.reshape(1, D)
    return _tc_compute(ent0, ent1, nwg, item_rows, adj,
                       a4, w1big, w2up, sumg, w3a, w3b, bias)


# speedup vs baseline: 3.0849x; 1.0758x over previous
import functools

import jax
import jax.numpy as jnp
import numpy as np
from jax import lax
from jax.experimental import pallas as pl
from jax.experimental.pallas import tpu as pltpu
from jax.experimental.pallas import tpu_sc as plsc

B, L, D = 256, 50, 64
S = 12
NUM_NODE = 50000
BL = B * L
LS = L * S
ALPHA = 0.2


def _sc_gather(inp_flat, item_flat, pos_flat, adj_flat, num_flat, embedding):
    info = plsc.get_sparse_core_info()
    nc, ns = info.num_cores, info.num_subcores
    nw = nc * ns
    per_w = BL // nw
    ch = 80
    n_chunks = per_w // ch
    mesh = plsc.VectorSubcoreMesh(core_axis_name="c", subcore_axis_name="s")

    @functools.partial(
        pl.kernel,
        out_type=(
            jax.ShapeDtypeStruct((BL, D), jnp.float32),
            jax.ShapeDtypeStruct((BL * S,), jnp.float32),
            jax.ShapeDtypeStruct((BL, D), jnp.float32),
            jax.ShapeDtypeStruct((BL * S, D), jnp.float32),
        ),
        mesh=mesh,
        compiler_params=pltpu.CompilerParams(use_tc_tiling_on_sc=False),
        scratch_types=[
            pltpu.VMEM((ch,), jnp.int32),
            pltpu.VMEM((ch, D), jnp.float32),
            pltpu.VMEM((ch * S,), jnp.int32),
            pltpu.VMEM((ch * S,), jnp.float32),
            pltpu.VMEM((ch * S,), jnp.int32),
            pltpu.VMEM((ch * S, D), jnp.float32),
            pltpu.VMEM((ch,), jnp.int32),
            pltpu.VMEM((ch, D), jnp.float32),
            pltpu.SemaphoreType.DMA,
            pltpu.SemaphoreType.DMA,
            pltpu.SemaphoreType.DMA,
            pltpu.SemaphoreType.DMA,
        ],
    )
    def gather_kernel(inp_hbm, item_hbm, pos_hbm, adjflat_hbm, numflat_hbm,
                      emb_hbm,
                      ent0_hbm, nw_hbm, item_emb_hbm, ent1_hbm,
                      idx_v, rows_v, pos_v, nw_v, nidx_v, nrows_v,
                      iidx_v, irows_v, sem0, sem1, sem2, sem3):
        wid = lax.axis_index("s") * nc + lax.axis_index("c")
        base = wid * per_w
        for c in range(n_chunks):
            off = base + c * ch
            pltpu.sync_copy(inp_hbm.at[pl.ds(off, ch)], idx_v)
            pltpu.sync_copy(item_hbm.at[pl.ds(off, ch)], iidx_v)
            pltpu.sync_copy(pos_hbm.at[pl.ds(off * S, ch * S)], pos_v)
            cp_nid = pltpu.async_copy(adjflat_hbm.at[pos_v], nidx_v, sem1)
            cp_e0 = pltpu.async_copy(emb_hbm.at[idx_v], rows_v, sem0)
            cp_nw = pltpu.async_copy(numflat_hbm.at[pos_v], nw_v, sem2)
            cp_it = pltpu.async_copy(emb_hbm.at[iidx_v], irows_v, sem3)
            cp_nid.wait()
            cp_e1 = pltpu.async_copy(emb_hbm.at[nidx_v], nrows_v, sem1)
            cp_e0.wait()
            pltpu.sync_copy(rows_v, ent0_hbm.at[pl.ds(off, ch)])
            cp_nw.wait()
            pltpu.sync_copy(nw_v, nw_hbm.at[pl.ds(off * S, ch * S)])
            cp_it.wait()
            pltpu.sync_copy(irows_v, item_emb_hbm.at[pl.ds(off, ch)])
            cp_e1.wait()
            pltpu.sync_copy(nrows_v, ent1_hbm.at[pl.ds(off * S, ch * S)])

    return gather_kernel(inp_flat, item_flat, pos_flat, adj_flat, num_flat,
                         embedding)


def _dot(x, y):
    return lax.dot_general(x, y, (((1,), (0,)), ((), ())),
                           preferred_element_type=jnp.float32)


def _dot_nt(x, y):
    return lax.dot_general(x, y, (((1,), (1,)), ((), ())),
                           preferred_element_type=jnp.float32)


def _leaky(x, slope):
    return jnp.where(x >= 0, x, slope * x)


def _tc_body(e0_ref, e1_ref, nw_ref, it_ref, adj_ref,
             a4_ref, w1big_ref, w2up_ref, sumg_ref, w3a_ref, w3b_ref,
             bias_ref, out_ref, *, bb):
    bl = bb * L
    a4 = a4_ref[...]
    w1big = w1big_ref[...]
    w2up = w2up_ref[...]
    sumg = sumg_ref[...]
    w3a = w3a_ref[...]
    w3b = w3b_ref[...]
    bias = bias_ref[...]
    e0 = e0_ref[...].reshape(bl, D)
    n2 = jnp.sum(e0 * e0, axis=1, keepdims=True)
    h = e0 / jnp.maximum(jnp.sqrt(n2), 1e-12)
    eks = [_leaky(_dot_nt(h * a4[k:k + 1, :], h), ALPHA) for k in range(4)]
    rows = []
    for i in range(bb):
        adji = adj_ref[i]
        li = jnp.full((L, L), -9e15, dtype=jnp.float32)
        for k in range(4):
            sub = eks[k][i * L:(i + 1) * L, i * L:(i + 1) * L]
            li = jnp.where(adji == k + 1, sub, li)
        pieces = []
        if i:
            pieces.append(jnp.full((L, i * L), -1.8e16, dtype=jnp.float32))
        pieces.append(li)
        if i < bb - 1:
            pieces.append(jnp.full((L, (bb - 1 - i) * L), -1.8e16,
                                   dtype=jnp.float32))
        rows.append(jnp.concatenate(pieces, axis=1) if len(pieces) > 1
                    else pieces[0])
    logits = jnp.concatenate(rows, axis=0) if bb > 1 else rows[0]
    m = jnp.max(logits, axis=1, keepdims=True)
    p = jnp.exp(logits - m)
    aw = p / jnp.sum(p, axis=1, keepdims=True)
    h_local = _dot(aw, h)
    sess = jnp.sum(it_ref[...], axis=1)
    sess_bl = jnp.broadcast_to(sess[:, None, :], (bb, L, D)).reshape(bl, D)
    e1w = e1_ref[...].reshape(bl, S * D)
    sessw = jnp.concatenate([sess_bl] * S, axis=1)
    nw12 = nw_ref[...].reshape(bl, S)
    zin = jnp.concatenate([e1w * sessw, nw12], axis=1)
    zl = _leaky(_dot(zin, w1big), 0.2)
    exu = jnp.exp(_dot(zl, w2up))
    accu = _dot(e1w * exu, sumg)
    denu = _dot(exu, sumg)
    neigh = accu / denu
    hg = jnp.maximum(_dot(e0, w3a) + _dot(neigh, w3b) + bias, 0.0)
    out_ref[...] = (h_local + hg).reshape(bb, L, D)


def _tc_compute(ent0, ent1, nw, item_rows, adj,
                a4, w1big, w2up, sumg, w3a, w3b, bias, bb=4):
    grid = (B // bb,)
    bspec = lambda shape: pl.BlockSpec((bb,) + shape, lambda g: (g,) + (0,) * len(shape))
    wspec = lambda shape: pl.BlockSpec(shape, lambda g: (0,) * len(shape))
    return pl.pallas_call(
        functools.partial(_tc_body, bb=bb),
        grid=grid,
        in_specs=[
            bspec((L, D)),
            pl.BlockSpec((bb * L * S * D // 128, 128),
                         lambda g: (g, 0)),
            bspec((L, S)),
            bspec((L, D)),
            bspec((L, L)),
            wspec((4, D)),
            wspec((S * D + S, S * D)),
            wspec((S * D, S * D)),
            wspec((S * D, D)),
            wspec((D, D)),
            wspec((D, D)),
            wspec((1, D)),
        ],
        out_specs=bspec((L, D)),
        out_shape=jax.ShapeDtypeStruct((B, L, D), jnp.float32),
    )(ent0, ent1, nw, item_rows, adj,
      a4, w1big, w2up, sumg, w3a, w3b, bias)


def kernel(inputs, adj, mask_item, item, adj_all, num, embedding,
           a0, a1, a2, a3, g_w1, g_w2, g_w3, g_bias):
    inp_flat = inputs.reshape(-1).astype(jnp.int32)
    item_flat = item.reshape(-1).astype(jnp.int32)
    pos_flat = (inp_flat[:, None]
                + jnp.arange(S, dtype=jnp.int32)[None, :] * NUM_NODE
                ).reshape(-1)
    adj_t = adj_all.astype(jnp.int32).T.reshape(-1)
    num_t = num.T.reshape(-1)
    ent0, nwg, item_rows, ent1 = _sc_gather(
        inp_flat, item_flat, pos_flat, adj_t, num_t, embedding)
    ent0 = ent0.reshape(B, L, D)
    ent1 = ent1.reshape(BL * S * D // 128, 128)
    nwg = nwg.reshape(B, L, S)
    maskf = mask_item.astype(jnp.float32)
    scale = (maskf / jnp.sum(maskf, axis=1, keepdims=True)).reshape(BL, 1)
    item_rows = (item_rows * scale).reshape(B, L, D)
    a4 = jnp.concatenate([a0, a1, a2, a3], axis=1).T
    w1a = g_w1[:D]
    w1b = g_w1[D:]
    eye_s = jnp.eye(S, dtype=jnp.float32)
    w1top = (eye_s[:, None, :, None] * w1a[None, :, None, :]
             ).reshape(S * D, S * D)
    w1bot = (eye_s[:, :, None] * w1b[0][None, None, :]).reshape(S, S * D)
    w1big = jnp.concatenate([w1top, w1bot], axis=0)
    w2up = jnp.broadcast_to(
        eye_s[:, None, :, None] * g_w2[:, 0][None, :, None, None],
        (S, D, S, D)).reshape(S * D, S * D)
    sumg = jnp.tile(jnp.eye(D, dtype=jnp.float32), (S, 1))
    w3a = g_w3[:D]
    w3b = g_w3[D:]
    bias = g_bias.reshape(1, D)
    return _tc_compute(ent0, ent1, nwg, item_rows, adj,
                       a4, w1big, w2up, sumg, w3a, w3b, bias)


# speedup vs baseline: 3.1633x; 1.0254x over previous
import functools

import jax
import jax.numpy as jnp
import numpy as np
from jax import lax
from jax.experimental import pallas as pl
from jax.experimental.pallas import tpu as pltpu
from jax.experimental.pallas import tpu_sc as plsc

B, L, D = 256, 50, 64
S = 12
NUM_NODE = 50000
BL = B * L
LS = L * S
ALPHA = 0.2


def _sc_gather(inp_flat, item_flat, pos_flat, adj_flat, num_flat, embedding):
    info = plsc.get_sparse_core_info()
    nc, ns = info.num_cores, info.num_subcores
    nw = nc * ns
    bln = inp_flat.shape[0]
    per_w = bln // nw
    n_chunks = 5
    ch = per_w // n_chunks
    mesh = plsc.VectorSubcoreMesh(core_axis_name="c", subcore_axis_name="s")

    @functools.partial(
        pl.kernel,
        out_type=(
            jax.ShapeDtypeStruct((bln, D), jnp.float32),
            jax.ShapeDtypeStruct((bln * S,), jnp.float32),
            jax.ShapeDtypeStruct((bln, D), jnp.float32),
            jax.ShapeDtypeStruct((bln * S, D), jnp.float32),
        ),
        mesh=mesh,
        compiler_params=pltpu.CompilerParams(use_tc_tiling_on_sc=False),
        scratch_types=[
            pltpu.VMEM((ch,), jnp.int32),
            pltpu.VMEM((ch, D), jnp.float32),
            pltpu.VMEM((ch * S,), jnp.int32),
            pltpu.VMEM((ch * S,), jnp.float32),
            pltpu.VMEM((ch * S,), jnp.int32),
            pltpu.VMEM((ch * S, D), jnp.float32),
            pltpu.VMEM((ch,), jnp.int32),
            pltpu.VMEM((ch, D), jnp.float32),
            pltpu.SemaphoreType.DMA,
            pltpu.SemaphoreType.DMA,
            pltpu.SemaphoreType.DMA,
            pltpu.SemaphoreType.DMA,
        ],
    )
    def gather_kernel(inp_hbm, item_hbm, pos_hbm, adjflat_hbm, numflat_hbm,
                      emb_hbm,
                      ent0_hbm, nw_hbm, item_emb_hbm, ent1_hbm,
                      idx_v, rows_v, pos_v, nw_v, nidx_v, nrows_v,
                      iidx_v, irows_v, sem0, sem1, sem2, sem3):
        wid = lax.axis_index("s") * nc + lax.axis_index("c")
        base = wid * per_w
        for c in range(n_chunks):
            off = base + c * ch
            pltpu.sync_copy(inp_hbm.at[pl.ds(off, ch)], idx_v)
            pltpu.sync_copy(item_hbm.at[pl.ds(off, ch)], iidx_v)
            pltpu.sync_copy(pos_hbm.at[pl.ds(off * S, ch * S)], pos_v)
            cp_nid = pltpu.async_copy(adjflat_hbm.at[pos_v], nidx_v, sem1)
            cp_e0 = pltpu.async_copy(emb_hbm.at[idx_v], rows_v, sem0)
            cp_nw = pltpu.async_copy(numflat_hbm.at[pos_v], nw_v, sem2)
            cp_it = pltpu.async_copy(emb_hbm.at[iidx_v], irows_v, sem3)
            cp_nid.wait()
            cp_e1 = pltpu.async_copy(emb_hbm.at[nidx_v], nrows_v, sem1)
            cp_e0.wait()
            pltpu.sync_copy(rows_v, ent0_hbm.at[pl.ds(off, ch)])
            cp_nw.wait()
            pltpu.sync_copy(nw_v, nw_hbm.at[pl.ds(off * S, ch * S)])
            cp_it.wait()
            pltpu.sync_copy(irows_v, item_emb_hbm.at[pl.ds(off, ch)])
            cp_e1.wait()
            pltpu.sync_copy(nrows_v, ent1_hbm.at[pl.ds(off * S, ch * S)])

    return gather_kernel(inp_flat, item_flat, pos_flat, adj_flat, num_flat,
                         embedding)


def _dot(x, y):
    return lax.dot_general(x, y, (((1,), (0,)), ((), ())),
                           preferred_element_type=jnp.float32)


def _dot_nt(x, y):
    return lax.dot_general(x, y, (((1,), (1,)), ((), ())),
                           preferred_element_type=jnp.float32)


def _leaky(x, slope):
    return jnp.where(x >= 0, x, slope * x)


def _tc_body(e0_ref, e1_ref, nw_ref, it_ref, adj_ref,
             a4_ref, w1big_ref, w2up_ref, sumg_ref, w3a_ref, w3b_ref,
             bias_ref, out_ref, *, bb):
    bl = bb * L
    a4 = a4_ref[...]
    w1big = w1big_ref[...]
    w2up = w2up_ref[...]
    sumg = sumg_ref[...]
    w3a = w3a_ref[...]
    w3b = w3b_ref[...]
    bias = bias_ref[...]
    e0 = e0_ref[...].reshape(bl, D)
    n2 = jnp.sum(e0 * e0, axis=1, keepdims=True)
    h = e0 / jnp.maximum(jnp.sqrt(n2), 1e-12)
    eks = [_leaky(_dot_nt(h * a4[k:k + 1, :], h), ALPHA) for k in range(4)]
    rows = []
    for i in range(bb):
        adji = adj_ref[i]
        li = jnp.full((L, L), -9e15, dtype=jnp.float32)
        for k in range(4):
            sub = eks[k][i * L:(i + 1) * L, i * L:(i + 1) * L]
            li = jnp.where(adji == k + 1, sub, li)
        pieces = []
        if i:
            pieces.append(jnp.full((L, i * L), -1.8e16, dtype=jnp.float32))
        pieces.append(li)
        if i < bb - 1:
            pieces.append(jnp.full((L, (bb - 1 - i) * L), -1.8e16,
                                   dtype=jnp.float32))
        rows.append(jnp.concatenate(pieces, axis=1) if len(pieces) > 1
                    else pieces[0])
    logits = jnp.concatenate(rows, axis=0) if bb > 1 else rows[0]
    m = jnp.max(logits, axis=1, keepdims=True)
    p = jnp.exp(logits - m)
    aw = p / jnp.sum(p, axis=1, keepdims=True)
    h_local = _dot(aw, h)
    sess = jnp.sum(it_ref[...], axis=1)
    sess_bl = jnp.broadcast_to(sess[:, None, :], (bb, L, D)).reshape(bl, D)
    e1w = e1_ref[...].reshape(bl, S * D)
    sessw = jnp.concatenate([sess_bl] * S, axis=1)
    nw12 = nw_ref[...].reshape(bl, S)
    zin = jnp.concatenate([e1w * sessw, nw12], axis=1)
    zl = _leaky(_dot(zin, w1big), 0.2)
    exu = jnp.exp(_dot(zl, w2up))
    accu = _dot(e1w * exu, sumg)
    denu = _dot(exu, sumg)
    neigh = accu / denu
    hg = jnp.maximum(_dot(e0, w3a) + _dot(neigh, w3b) + bias, 0.0)
    out_ref[...] = (h_local + hg).reshape(bb, L, D)


def _tc_compute(ent0, ent1, nw, item_rows, adj,
                a4, w1big, w2up, sumg, w3a, w3b, bias, bb=4):
    nb = adj.shape[0]
    grid = (nb // bb,)
    bspec = lambda shape: pl.BlockSpec((bb,) + shape, lambda g: (g,) + (0,) * len(shape))
    wspec = lambda shape: pl.BlockSpec(shape, lambda g: (0,) * len(shape))
    return pl.pallas_call(
        functools.partial(_tc_body, bb=bb),
        grid=grid,
        in_specs=[
            bspec((L, D)),
            pl.BlockSpec((bb * L * S * D // 128, 128),
                         lambda g: (g, 0)),
            bspec((L, S)),
            bspec((L, D)),
            bspec((L, L)),
            wspec((4, D)),
            wspec((S * D + S, S * D)),
            wspec((S * D, S * D)),
            wspec((S * D, D)),
            wspec((D, D)),
            wspec((D, D)),
            wspec((1, D)),
        ],
        out_specs=bspec((L, D)),
        out_shape=jax.ShapeDtypeStruct((nb, L, D), jnp.float32),
    )(ent0, ent1, nw, item_rows, adj,
      a4, w1big, w2up, sumg, w3a, w3b, bias)


def kernel(inputs, adj, mask_item, item, adj_all, num, embedding,
           a0, a1, a2, a3, g_w1, g_w2, g_w3, g_bias):
    inp_flat = inputs.reshape(-1).astype(jnp.int32)
    item_flat = item.reshape(-1).astype(jnp.int32)
    pos_flat = (inp_flat[:, None]
                + jnp.arange(S, dtype=jnp.int32)[None, :] * NUM_NODE
                ).reshape(-1)
    adj_t = adj_all.astype(jnp.int32).T.reshape(-1)
    num_t = num.T.reshape(-1)
    maskf = mask_item.astype(jnp.float32)
    scale = (maskf / jnp.sum(maskf, axis=1, keepdims=True)).reshape(BL, 1)
    a4 = jnp.concatenate([a0, a1, a2, a3], axis=1).T
    w1a = g_w1[:D]
    w1b = g_w1[D:]
    eye_s = jnp.eye(S, dtype=jnp.float32)
    w1top = (eye_s[:, None, :, None] * w1a[None, :, None, :]
             ).reshape(S * D, S * D)
    w1bot = (eye_s[:, :, None] * w1b[0][None, None, :]).reshape(S, S * D)
    w1big = jnp.concatenate([w1top, w1bot], axis=0)
    w2up = jnp.broadcast_to(
        eye_s[:, None, :, None] * g_w2[:, 0][None, :, None, None],
        (S, D, S, D)).reshape(S * D, S * D)
    sumg = jnp.tile(jnp.eye(D, dtype=jnp.float32), (S, 1))
    w3a = g_w3[:D]
    w3b = g_w3[D:]
    bias = g_bias.reshape(1, D)
    npiece = 2
    hb = B // npiece
    outs = []
    for p in range(npiece):
        sl = slice(p * hb * L, (p + 1) * hb * L)
        sl12 = slice(p * hb * LS, (p + 1) * hb * LS)
        e0h, nwh, ith, e1h = _sc_gather(
            inp_flat[sl], item_flat[sl], pos_flat[sl12], adj_t, num_t,
            embedding)
        e0h = e0h.reshape(hb, L, D)
        e1h = e1h.reshape(hb * L * S * D // 128, 128)
        nwh = nwh.reshape(hb, L, S)
        ith = (ith * scale[sl]).reshape(hb, L, D)
        outs.append(_tc_compute(e0h, e1h, nwh, ith, adj[p * hb:(p + 1) * hb],
                                a4, w1big, w2up, sumg, w3a, w3b, bias))
    return jnp.concatenate(outs, axis=0)
